# baseline (speedup 1.0000x reference)
import functools

import numpy as np
import jax
import jax.numpy as jnp
from jax import lax
from jax.experimental import pallas as pl
from jax.experimental.pallas import tpu as pltpu
from jax.experimental.pallas import tpu_sc as plsc

N = 10000
D = 128
E = 320000
STEPS = 1000
HID = 256
TDIM = 128

_betas = np.linspace(1e-4, 0.02, STEPS)
_acp = np.cumprod(1.0 - _betas)
_SQRT_ACP = np.sqrt(_acp).astype(np.float32)
_SQRT_OMACP = np.sqrt(1.0 - _acp).astype(np.float32)
_FREQS = np.exp(
    -np.log(10000.0) * np.arange(TDIM // 2) / (TDIM // 2)
).astype(np.float32)[None, :]

_GDN = lax.GatherDimensionNumbers(
    offset_dims=(), collapsed_slice_dims=(0,), start_index_map=(0,)
)

_CH = 128
_NB = 3
_NCHT = -(-E // _CH)
_NCHT += (-_NCHT) % _NB
_EPAD = _NCHT * _CH
_NQ = _NCHT // _NB
_NQ_BASE = _NQ // 32
_NQ_EXTRA = _NQ - 32 * _NQ_BASE
_NSUB = 16
_TRASH = 0
_RSTEP = 624
_RSPAN = 640


def _sc_hop(src, pk3, zblk):
    mesh = plsc.VectorSubcoreMesh(core_axis_name="c", subcore_axis_name="s")

    @functools.partial(
        pl.kernel,
        out_type=jax.ShapeDtypeStruct((2, N, D), jnp.float32),
        mesh=mesh,
        scratch_types=[
            [pltpu.VMEM((3, _CH), jnp.int32) for _ in range(_NB)],
            [pltpu.VMEM((_CH, D), jnp.float32) for _ in range(_NB)],
            [pltpu.VMEM((_CH,), jnp.int32) for _ in range(_NB)],
            pltpu.VMEM_SHARED((N, D), jnp.float32),
            [pltpu.SemaphoreType.DMA for _ in range(_NB)],
            [pltpu.SemaphoreType.DMA for _ in range(_NB)],
            [pltpu.SemaphoreType.DMA for _ in range(_NB)],
        ],
    )
    def hop(src_hbm, pk_hbm, z_hbm, out_hbm,
            pk, rows, rbx, acc_sh, ssem, gsem, csem):
        cid = lax.axis_index("c")
        sid = lax.axis_index("s")
        wid = sid * 2 + cid
        pltpu.sync_copy(z_hbm, acc_sh.at[pl.ds(sid * _RSTEP, _RSPAN)])
        plsc.subcore_barrier()

        nq = _NQ_BASE + jnp.where(wid < _NQ_EXTRA, 1, 0)
        q0 = _NQ_BASE * wid + jnp.minimum(wid, _NQ_EXTRA)
        c0 = _NB * q0
        nc = _NB * nq

        def sup_issue(gc, u):
            pltpu.async_copy(
                pk_hbm.at[:, pl.ds(gc * _CH, _CH)], pk[u], ssem[u]
            )

        def sup_wait(gc, u):
            pltpu.make_async_copy(
                pk_hbm.at[:, pl.ds(gc * _CH, _CH)], pk[u], ssem[u]
            ).wait()

        def g_issue(u, p):
            pltpu.async_copy(src_hbm.at[pk[u].at[0]], rows[p], gsem[p])

        def g_wait(u, p):
            pltpu.make_async_copy(src_hbm.at[pk[u].at[0]], rows[p], gsem[p]).wait()

        def sc_wait(p):
            pltpu.make_async_copy(rows[p], acc_sh.at[rbx[p]], csem[p]).wait()

        def scale(u, p):
            def grp(g, carry2):
                vals = lax.bitcast_convert_type(
                    pk[u][2, pl.ds(g * 16, 16)], jnp.float32
                )
                for l in range(16):
                    bc = lax.gather(
                        vals,
                        jnp.full((16, 1), l, jnp.int32),
                        _GDN,
                        slice_sizes=(1,),
                        mode=lax.GatherScatterMode.PROMISE_IN_BOUNDS,
                    )
                    e = g * 16 + l
                    for j in range(D // 16):
                        sl = pl.ds(j * 16, 16)
                        rows[p][e, sl] = rows[p][e, sl] * bc
                return carry2

            lax.fori_loop(0, _CH // 16, grp, 0)

        for u in range(_NB):
            sup_issue(c0 + u, u)
        sup_wait(c0, 0)
        g_issue(0, 0)
        sup_wait(c0 + 1, 1)
        g_issue(1, 1)

        def triad(k, carry):
            for b in range(_NB):
                lc = _NB * k + b
                gc = c0 + lc
                bn = (b + 2) % _NB
                g_wait(b, b)

                @pl.when(lc + 2 < nc)
                def _nx():
                    if b == 0:
                        @pl.when(k >= 1)
                        def _dr():
                            sc_wait(bn)
                    else:
                        sc_wait(bn)
                    sup_wait(gc + 2, bn)
                    g_issue(bn, bn)

                scale(b, b)
                for g in range(_CH // 16):
                    sl = pl.ds(g * 16, 16)
                    rbx[b][sl] = pk[b][1, sl]
                pltpu.async_copy(rows[b], acc_sh.at[rbx[b]], csem[b], add=True)

                @pl.when(lc + _NB < nc)
                def _sup():
                    sup_issue(gc + _NB, b)
            return carry

        lax.fori_loop(0, nq, triad, 0)
        for b in range(_NB):
            sc_wait(b)

        plsc.subcore_barrier()
        pltpu.sync_copy(
            acc_sh.at[pl.ds(sid * _RSTEP, _RSPAN)],
            out_hbm.at[cid, pl.ds(sid * _RSTEP, _RSPAN)],
        )

    return hop(src, pk3, zblk)


_BR = 1000
_NG = N // _BR


def _tc_body(vf_ref, mr_ref, nz_ref, tf_ref, sc_ref, sa_ref, so_ref, t_ref,
             fr_ref, W1_ref, b1_ref, W2_ref, b2_ref, out_ref):
    i = pl.program_id(0)
    m1 = jnp.where(mr_ref[...] < 0.5, 0.0, 1.0)
    xt = sa_ref[...] * (vf_ref[...] * m1) + so_ref[...] * nz_ref[...]
    sc = sc_ref[0] + sc_ref[1]
    targs = t_ref[...] * fr_ref[...]
    w1 = W1_ref[...]
    h = (
        jnp.dot(xt, w1[0:128], preferred_element_type=jnp.float32)
        + jnp.dot(sc, w1[128:256], preferred_element_type=jnp.float32)
        + jnp.dot(tf_ref[...], w1[256:384], preferred_element_type=jnp.float32)
        + jnp.dot(jnp.cos(targs), w1[384:448], preferred_element_type=jnp.float32)
        + jnp.dot(jnp.sin(targs), w1[448:512], preferred_element_type=jnp.float32)
        + b1_ref[...]
    )
    h = jnp.maximum(h, 0.0)
    pred = jnp.dot(h, W2_ref[...], preferred_element_type=jnp.float32) + b2_ref[...]
    dres = nz_ref[...] - pred
    s = jnp.reshape(jnp.sum(dres * dres), (1, 1))

    @pl.when(i == 0)
    def _init():
        out_ref[...] = jnp.zeros((1, 1), jnp.float32)

    out_ref[...] += s

    @pl.when(i == _NG - 1)
    def _fin():
        out_ref[...] = out_ref[...] * (1.0 / (N * D))


def _tc_tail(v_feat, mask_rand, noise, t_feat, sc2, sa, so, t_f32, W1, b1, W2, b2):
    row_blk = lambda: pl.BlockSpec((_BR, D), lambda i: (i, 0))
    col1 = lambda: pl.BlockSpec((_BR, 1), lambda i: (i, 0))
    return pl.pallas_call(
        _tc_body,
        grid=(_NG,),
        in_specs=[
            row_blk(),
            col1(),
            row_blk(),
            row_blk(),
            pl.BlockSpec((2, _BR, D), lambda i: (0, i, 0)),
            col1(),
            col1(),
            col1(),
            pl.BlockSpec((1, TDIM // 2), lambda i: (0, 0)),
            pl.BlockSpec((3 * D + TDIM, HID), lambda i: (0, 0)),
            pl.BlockSpec((1, HID), lambda i: (0, 0)),
            pl.BlockSpec((HID, D), lambda i: (0, 0)),
            pl.BlockSpec((1, D), lambda i: (0, 0)),
        ],
        out_specs=pl.BlockSpec((1, 1), lambda i: (0, 0)),
        out_shape=jax.ShapeDtypeStruct((1, 1), jnp.float32),
    )(v_feat, mask_rand, noise, t_feat, sc2, sa, so, t_f32, _FREQS,
      W1, b1, W2, b2)


def kernel(t_feat_online, v_feat_online, adj_val, mask_rand, noise,
           W1, b1, W2, b2, adj_row, adj_col, t):
    npad = _EPAD - E
    pk3 = jnp.stack(
        [
            jnp.concatenate(
                [adj_col.astype(jnp.int32), jnp.zeros((npad,), jnp.int32)]
            ),
            jnp.concatenate(
                [adj_row.astype(jnp.int32), jnp.full((npad,), _TRASH, jnp.int32)]
            ),
            jnp.concatenate(
                [
                    lax.bitcast_convert_type(adj_val, jnp.int32),
                    jnp.zeros((npad,), jnp.int32),
                ]
            ),
        ],
        axis=0,
    )
    zblk = jnp.zeros((_RSPAN, D), jnp.float32)
    p1 = _sc_hop(t_feat_online, pk3, zblk)
    s1 = p1[0] + p1[1]
    p2 = _sc_hop(s1, pk3, zblk)
    sa = jnp.take(jnp.asarray(_SQRT_ACP), t)[:, None]
    so = jnp.take(jnp.asarray(_SQRT_OMACP), t)[:, None]
    res = _tc_tail(
        v_feat_online, mask_rand[:, None], noise, t_feat_online, p2,
        sa, so, t.astype(jnp.float32)[:, None], W1, b1[None, :], W2, b2[None, :],
    )
    return res[0, 0]


# speedup vs baseline: 1.0685x; 1.0685x over previous
import functools

import numpy as np
import jax
import jax.numpy as jnp
from jax import lax
from jax.experimental import pallas as pl
from jax.experimental.pallas import tpu as pltpu
from jax.experimental.pallas import tpu_sc as plsc

N = 10000
D = 128
E = 320000
STEPS = 1000
HID = 256
TDIM = 128

_betas = np.linspace(1e-4, 0.02, STEPS)
_acp = np.cumprod(1.0 - _betas)
_SQRT_ACP = np.sqrt(_acp).astype(np.float32)
_SQRT_OMACP = np.sqrt(1.0 - _acp).astype(np.float32)
_FREQS = np.exp(
    -np.log(10000.0) * np.arange(TDIM // 2) / (TDIM // 2)
).astype(np.float32)[None, :]

_GDN = lax.GatherDimensionNumbers(
    offset_dims=(), collapsed_slice_dims=(0,), start_index_map=(0,)
)

_CH = 128
_NB = 3
_NCHT = -(-E // _CH)
_NCHT += (-_NCHT) % _NB
_EPAD = _NCHT * _CH
_NQ = _NCHT // _NB
_NQ_BASE = _NQ // 32
_NQ_EXTRA = _NQ - 32 * _NQ_BASE
_NSUB = 16
_TRASH = 0
_RSTEP = 624
_RSPAN = 640


def _sc_hop(src, col2d, row2d, val2d, zblk):
    mesh = plsc.VectorSubcoreMesh(core_axis_name="c", subcore_axis_name="s")

    @functools.partial(
        pl.kernel,
        out_type=jax.ShapeDtypeStruct((2, N, D), jnp.float32),
        mesh=mesh,
        scratch_types=[
            [pltpu.VMEM((_CH,), jnp.int32) for _ in range(_NB)],
            [pltpu.VMEM((_CH,), jnp.int32) for _ in range(_NB)],
            [pltpu.VMEM((_CH,), jnp.float32) for _ in range(_NB)],
            [pltpu.VMEM((_CH, D), jnp.float32) for _ in range(_NB)],
            [pltpu.VMEM((_CH,), jnp.int32) for _ in range(_NB)],
            pltpu.VMEM_SHARED((N, D), jnp.float32),
            [pltpu.SemaphoreType.DMA for _ in range(_NB)],
            [pltpu.SemaphoreType.DMA for _ in range(_NB)],
            [pltpu.SemaphoreType.DMA for _ in range(_NB)],
        ],
    )
    def hop(src_hbm, col_hbm, row_hbm, val_hbm, z_hbm, out_hbm,
            cb, rb, vb, rows, rbx, acc_sh, ssem, gsem, csem):
        cid = lax.axis_index("c")
        sid = lax.axis_index("s")
        wid = sid * 2 + cid

        nq = _NQ_BASE + jnp.where(wid < _NQ_EXTRA, 1, 0)
        q0 = _NQ_BASE * wid + jnp.minimum(wid, _NQ_EXTRA)
        c0 = _NB * q0
        nc = _NB * nq

        def sup_issue(gc, u):
            pltpu.async_copy(col_hbm.at[gc], cb[u], ssem[u])
            pltpu.async_copy(row_hbm.at[gc], rb[u], ssem[u])
            pltpu.async_copy(val_hbm.at[gc], vb[u], ssem[u])

        def sup_wait(gc, u):
            pltpu.make_async_copy(col_hbm.at[gc], cb[u], ssem[u]).wait()
            pltpu.make_async_copy(row_hbm.at[gc], rb[u], ssem[u]).wait()
            pltpu.make_async_copy(val_hbm.at[gc], vb[u], ssem[u]).wait()

        def g_issue(u, p):
            pltpu.async_copy(src_hbm.at[cb[u]], rows[p], gsem[p])

        def g_wait(u, p):
            pltpu.make_async_copy(src_hbm.at[cb[u]], rows[p], gsem[p]).wait()

        def sc_wait(p):
            pltpu.make_async_copy(rows[p], acc_sh.at[rbx[p]], csem[p]).wait()

        def scale(u, p):
            def grp(g, carry2):
                vals = vb[u][pl.ds(g * 16, 16)]
                for l in range(16):
                    bc = lax.gather(
                        vals,
                        jnp.full((16, 1), l, jnp.int32),
                        _GDN,
                        slice_sizes=(1,),
                        mode=lax.GatherScatterMode.PROMISE_IN_BOUNDS,
                    )
                    e = g * 16 + l
                    for j in range(D // 16):
                        sl = pl.ds(j * 16, 16)
                        rows[p][e, sl] = rows[p][e, sl] * bc
                return carry2

            lax.fori_loop(0, _CH // 16, grp, 0)

        for u in range(_NB):
            sup_issue(c0 + u, u)
        sup_wait(c0, 0)
        g_issue(0, 0)
        sup_wait(c0 + 1, 1)
        g_issue(1, 1)
        pltpu.sync_copy(z_hbm, acc_sh.at[pl.ds(sid * _RSTEP, _RSPAN)])
        plsc.subcore_barrier()

        def triad(k, carry):
            for b in range(_NB):
                lc = _NB * k + b
                gc = c0 + lc
                bn = (b + 2) % _NB
                g_wait(b, b)

                @pl.when(lc + 2 < nc)
                def _nx():
                    if b == 0:
                        @pl.when(k >= 1)
                        def _dr():
                            sc_wait(bn)
                    else:
                        sc_wait(bn)
                    sup_wait(gc + 2, bn)
                    g_issue(bn, bn)

                scale(b, b)
                for g in range(_CH // 16):
                    sl = pl.ds(g * 16, 16)
                    rbx[b][sl] = rb[b][sl]
                pltpu.async_copy(rows[b], acc_sh.at[rbx[b]], csem[b], add=True)

                @pl.when(lc + _NB < nc)
                def _sup():
                    sup_issue(gc + _NB, b)
            return carry

        lax.fori_loop(0, nq, triad, 0)
        for b in range(_NB):
            sc_wait(b)

        plsc.subcore_barrier()
        pltpu.sync_copy(
            acc_sh.at[pl.ds(sid * _RSTEP, _RSPAN)],
            out_hbm.at[cid, pl.ds(sid * _RSTEP, _RSPAN)],
        )

    return hop(src, col2d, row2d, val2d, zblk)


_BR = 1000
_NG = N // _BR


def _tcA_body(vf_ref, mr_ref, nz_ref, tf_ref, sa_ref, so_ref, t_ref,
              fr_ref, W1_ref, b1_ref, out_ref):
    m1 = jnp.where(mr_ref[...] < 0.5, 0.0, 1.0)
    xt = sa_ref[...] * (vf_ref[...] * m1) + so_ref[...] * nz_ref[...]
    targs = t_ref[...] * fr_ref[...]
    w1 = W1_ref[...]
    out_ref[...] = (
        jnp.dot(xt, w1[0:128], preferred_element_type=jnp.float32)
        + jnp.dot(tf_ref[...], w1[256:384], preferred_element_type=jnp.float32)
        + jnp.dot(jnp.cos(targs), w1[384:448], preferred_element_type=jnp.float32)
        + jnp.dot(jnp.sin(targs), w1[448:512], preferred_element_type=jnp.float32)
        + b1_ref[...]
    )


def _tc_pre(v_feat, mask_rand, noise, sa, so, t_f32, t_feat, W1, b1):
    row_blk = lambda: pl.BlockSpec((_BR, D), lambda i: (i, 0))
    col1 = lambda: pl.BlockSpec((_BR, 1), lambda i: (i, 0))
    return pl.pallas_call(
        _tcA_body,
        grid=(_NG,),
        in_specs=[
            row_blk(),
            col1(),
            row_blk(),
            row_blk(),
            col1(),
            col1(),
            col1(),
            pl.BlockSpec((1, TDIM // 2), lambda i: (0, 0)),
            pl.BlockSpec((3 * D + TDIM, HID), lambda i: (0, 0)),
            pl.BlockSpec((1, HID), lambda i: (0, 0)),
        ],
        out_specs=pl.BlockSpec((_BR, HID), lambda i: (i, 0)),
        out_shape=jax.ShapeDtypeStruct((N, HID), jnp.float32),
    )(v_feat, mask_rand, noise, t_feat, sa, so, t_f32, _FREQS, W1, b1)


def _tcB_body(hp_ref, sc_ref, nz_ref, W1_ref, W2_ref, b2_ref, out_ref):
    i = pl.program_id(0)
    sc = sc_ref[0] + sc_ref[1]
    h = hp_ref[...] + jnp.dot(
        sc, W1_ref[...], preferred_element_type=jnp.float32
    )
    h = jnp.maximum(h, 0.0)
    pred = jnp.dot(h, W2_ref[...], preferred_element_type=jnp.float32) + b2_ref[...]
    dres = nz_ref[...] - pred
    s = jnp.reshape(jnp.sum(dres * dres), (1, 1))

    @pl.when(i == 0)
    def _init():
        out_ref[...] = jnp.zeros((1, 1), jnp.float32)

    out_ref[...] += s

    @pl.when(i == _NG - 1)
    def _fin():
        out_ref[...] = out_ref[...] * (1.0 / (N * D))


def _tc_tail(hpre, sc2, noise, W1b, W2, b2):
    return pl.pallas_call(
        _tcB_body,
        grid=(_NG,),
        in_specs=[
            pl.BlockSpec((_BR, HID), lambda i: (i, 0)),
            pl.BlockSpec((2, _BR, D), lambda i: (0, i, 0)),
            pl.BlockSpec((_BR, D), lambda i: (i, 0)),
            pl.BlockSpec((D, HID), lambda i: (0, 0)),
            pl.BlockSpec((HID, D), lambda i: (0, 0)),
            pl.BlockSpec((1, D), lambda i: (0, 0)),
        ],
        out_specs=pl.BlockSpec((1, 1), lambda i: (0, 0)),
        out_shape=jax.ShapeDtypeStruct((1, 1), jnp.float32),
    )(hpre, sc2, noise, W1b, W2, b2)


def kernel(t_feat_online, v_feat_online, adj_val, mask_rand, noise,
           W1, b1, W2, b2, adj_row, adj_col, t):
    npad = _EPAD - E
    col2d = jnp.concatenate(
        [adj_col.astype(jnp.int32), jnp.zeros((npad,), jnp.int32)]
    ).reshape(_NCHT, _CH)
    row2d = jnp.concatenate(
        [adj_row.astype(jnp.int32), jnp.full((npad,), _TRASH, jnp.int32)]
    ).reshape(_NCHT, _CH)
    val2d = jnp.concatenate(
        [adj_val, jnp.zeros((npad,), jnp.float32)]
    ).reshape(_NCHT, _CH)
    zblk = jnp.zeros((_RSPAN, D), jnp.float32)
    p1 = _sc_hop(t_feat_online, col2d, row2d, val2d, zblk)
    s1 = p1[0] + p1[1]
    p2 = _sc_hop(s1, col2d, row2d, val2d, zblk)
    sa = jnp.take(jnp.asarray(_SQRT_ACP), t)[:, None]
    so = jnp.take(jnp.asarray(_SQRT_OMACP), t)[:, None]
    hpre = _tc_pre(
        v_feat_online, mask_rand[:, None], noise, sa, so,
        t.astype(jnp.float32)[:, None], t_feat_online, W1, b1[None, :],
    )
    res = _tc_tail(hpre, p2, noise, W1[D : 2 * D], W2, b2[None, :])
    return res[0, 0]
